# Initial kernel scaffold; baseline (speedup 1.0000x reference)
import jax
import jax.numpy as jnp
from jax.experimental import pallas as pl


def kernel(user_ids, content_ids, user_table, content_table, W0, b0, g0, beta0, rm0, rv0, W1, b1, g1, beta1, rm1, rv1, W2, b2, g2, beta2, rm2, rv2, W3, b3):
    raise NotImplementedError("write your pallas kernel here")


# speedup vs baseline: 4.4418x; 4.4418x over previous
import functools

import jax
import jax.numpy as jnp
from jax import lax
from jax.experimental import pallas as pl
from jax.experimental.pallas import tpu as pltpu
from jax.experimental.pallas import tpu_sc as plsc

BATCH = 16384
EMB = 128
EPS = 1e-5

NC = 2
NS = 16
NW = NC * NS
RPW = BATCH // NW
CH = 128
NCH = RPW // CH


def _gather_body(uid_hbm, cid_hbm, utab_hbm, ctab_hbm, ue_hbm, ce_hbm,
                 idx_v, rows_a, rows_b, sem_a, sem_b):
    wid = lax.axis_index("s") * NC + lax.axis_index("c")
    base = wid * RPW
    for tab_hbm, ids_hbm, out_hbm in ((utab_hbm, uid_hbm, ue_hbm),
                                      (ctab_hbm, cid_hbm, ce_hbm)):
        pltpu.sync_copy(ids_hbm.at[pl.ds(wid * NCH, NCH)], idx_v)
        pltpu.async_copy(tab_hbm.at[idx_v.at[0]], rows_a, sem_a)
        for j in range(NCH):
            cur, nxt = (rows_a, rows_b) if j % 2 == 0 else (rows_b, rows_a)
            cur_sem, nxt_sem = (sem_a, sem_b) if j % 2 == 0 else (sem_b, sem_a)
            if j + 1 < NCH:
                pltpu.async_copy(tab_hbm.at[idx_v.at[j + 1]], nxt, nxt_sem)
            pltpu.make_async_copy(tab_hbm.at[idx_v.at[j]], cur, cur_sem).wait()
            pltpu.sync_copy(cur, out_hbm.at[pl.ds(base + j * CH, CH)])


@functools.cache
def _gather():
    return pl.kernel(
        _gather_body,
        out_type=(jax.ShapeDtypeStruct((BATCH, EMB), jnp.float32),
                  jax.ShapeDtypeStruct((BATCH, EMB), jnp.float32)),
        mesh=plsc.VectorSubcoreMesh(core_axis_name="c", subcore_axis_name="s",
                                    num_cores=NC, num_subcores=NS),
        scratch_types=(pltpu.VMEM((NCH, CH), jnp.int32),
                       pltpu.VMEM((CH, EMB), jnp.float32),
                       pltpu.VMEM((CH, EMB), jnp.float32),
                       pltpu.SemaphoreType.DMA,
                       pltpu.SemaphoreType.DMA),
    )


BB = 2048


def _mlp_body(ue, ce, w0a, w0b, b0, w1, b1, w2, b2, w3, b3, out):
    z0 = (jnp.dot(ue[...], w0a[...], preferred_element_type=jnp.float32)
          + jnp.dot(ce[...], w0b[...], preferred_element_type=jnp.float32)
          + b0[...])
    h0 = jnp.maximum(z0, 0.0)
    z1 = jnp.dot(h0, w1[...], preferred_element_type=jnp.float32) + b1[...]
    h1 = jnp.maximum(z1, 0.0)
    z2 = jnp.dot(h1, w2[...], preferred_element_type=jnp.float32) + b2[...]
    h2 = jnp.maximum(z2, 0.0)
    z3 = jnp.sum(h2 * w3[...], axis=1) + b3[0, 0]
    out[...] = 1.0 / (1.0 + jnp.exp(-z3))


def _full(shape):
    return pl.BlockSpec(shape, lambda i: (0,) * len(shape))


_mlp = pl.pallas_call(
    _mlp_body,
    grid=(BATCH // BB,),
    in_specs=[
        pl.BlockSpec((BB, EMB), lambda i: (i, 0)),
        pl.BlockSpec((BB, EMB), lambda i: (i, 0)),
        _full((EMB, 256)),
        _full((EMB, 256)),
        _full((1, 256)),
        _full((256, 128)),
        _full((1, 128)),
        _full((128, 64)),
        _full((1, 64)),
        _full((1, 64)),
        _full((1, 1)),
    ],
    out_specs=pl.BlockSpec((BB,), lambda i: (i,)),
    out_shape=jax.ShapeDtypeStruct((BATCH,), jnp.float32),
    compiler_params=pltpu.CompilerParams(
        dimension_semantics=("arbitrary",)),
)


def kernel(user_ids, content_ids, user_table, content_table,
           W0, b0, g0, beta0, rm0, rv0,
           W1, b1, g1, beta1, rm1, rv1,
           W2, b2, g2, beta2, rm2, rv2,
           W3, b3):
    uid = user_ids.astype(jnp.int32).reshape(BATCH // CH, CH)
    cid = content_ids.astype(jnp.int32).reshape(BATCH // CH, CH)
    ue, ce = _gather()(uid, cid, user_table, content_table)

    s0 = g0 * lax.rsqrt(rv0 + EPS)
    t0 = beta0 - rm0 * s0
    s1 = g1 * lax.rsqrt(rv1 + EPS)
    t1 = beta1 - rm1 * s1
    s2 = g2 * lax.rsqrt(rv2 + EPS)
    t2 = beta2 - rm2 * s2

    w0a = W0[:, :EMB].T
    w0b = W0[:, EMB:].T
    w1 = (W1 * s0[None, :]).T
    b1f = b1 + W1 @ t0
    w2 = (W2 * s1[None, :]).T
    b2f = b2 + W2 @ t1
    w3 = W3 * s2[None, :]
    b3f = (b3 + W3 @ t2).reshape(1, 1)

    return _mlp(ue, ce, w0a, w0b, b0.reshape(1, -1),
                w1, b1f.reshape(1, -1), w2, b2f.reshape(1, -1),
                w3, b3f)
